# baseline (speedup 1.0000x reference)
import functools

import jax
import jax.numpy as jnp
from jax import lax
from jax.experimental import pallas as pl
from jax.experimental.pallas import tpu as pltpu
from jax.experimental.pallas import tpu_sc as plsc

DIM = 64
NUM_WORKERS = 32
CHUNK = 128

def _gather_body(table_hbm, idx_hbm, out_hbm, idx_v, rows_v, sem):
    n_chunk = idx_hbm.shape[1]
    wid = lax.axis_index("s") * 2 + lax.axis_index("c")
    base = wid * (n_chunk * CHUNK)
    pltpu.sync_copy(idx_hbm.at[wid], idx_v)

    def body(j, carry):
        pltpu.async_copy(table_hbm.at[idx_v.at[j]], rows_v, sem).wait()
        pltpu.sync_copy(rows_v, out_hbm.at[pl.ds(base + j * CHUNK, CHUNK)])
        return carry

    lax.fori_loop(0, n_chunk, body, 0)


@jax.jit
def _run(idx, table):
    n_total = idx.shape[0] * idx.shape[1] * idx.shape[2]
    mesh = plsc.VectorSubcoreMesh(core_axis_name="c", subcore_axis_name="s")
    k = functools.partial(
        pl.kernel,
        mesh=mesh,
        compiler_params=pltpu.CompilerParams(use_tc_tiling_on_sc=False),
        out_type=jax.ShapeDtypeStruct((n_total, DIM), jnp.float32),
        scratch_types=[
            pltpu.VMEM((idx.shape[1], CHUNK), jnp.int32),
            pltpu.VMEM((CHUNK, DIM), jnp.float32),
            pltpu.SemaphoreType.DMA,
        ],
    )(_gather_body)
    return k(table, idx)


def kernel(input_data, embeddings_matrix):
    b, h = input_data.shape
    n_total = b * h
    assert n_total % (NUM_WORKERS * CHUNK) == 0
    n_chunk = n_total // (NUM_WORKERS * CHUNK)
    idx = input_data.astype(jnp.int32).reshape(NUM_WORKERS, n_chunk, CHUNK)
    out = _run(idx, embeddings_matrix)
    return out.reshape(b, h, DIM)


# speedup vs baseline: 1.0477x; 1.0477x over previous
import functools

import jax
import jax.numpy as jnp
from jax import lax
from jax.experimental import pallas as pl
from jax.experimental.pallas import tpu as pltpu
from jax.experimental.pallas import tpu_sc as plsc

DIM = 64
NUM_WORKERS = 32
CHUNK = 128
NBUF = 10
DEPTH = 5


def _gather_body(table_hbm, idx_hbm, out_hbm, idx_v, *scratch):
    n_chunk = idx_hbm.shape[1]
    n_group = n_chunk // NBUF
    rows = scratch[0:NBUF]
    gsem = scratch[NBUF:2 * NBUF]
    wsem = scratch[2 * NBUF:3 * NBUF]

    wid = lax.axis_index("s") * 2 + lax.axis_index("c")
    base = wid * (n_chunk * CHUNK)
    pltpu.sync_copy(idx_hbm.at[wid], idx_v)

    def gather(c, b):
        return pltpu.make_async_copy(table_hbm.at[idx_v.at[c]], rows[b], gsem[b])

    def write(c, b):
        dst = out_hbm.at[pl.ds(base + c * CHUNK, CHUNK)]
        return pltpu.make_async_copy(rows[b], dst, wsem[b])

    def visit(c, b, drain, refill):
        gather(c, b).wait()
        write(c, b).start()
        nb = (b + DEPTH) % NBUF
        if drain:
            write(0, nb).wait()
        if refill:
            gather(c + DEPTH, nb).start()

    for b in range(DEPTH):
        gather(b, b).start()
    for b in range(NBUF):
        visit(b, b, drain=(b >= DEPTH), refill=True)

    def steady(k, carry):
        for b in range(NBUF):
            visit(k * NBUF + b, b, drain=True, refill=True)
        return carry

    lax.fori_loop(1, n_group - 1, steady, 0)

    for b in range(NBUF):
        c = (n_group - 1) * NBUF + b
        visit(c, b, drain=(b < DEPTH), refill=(b < DEPTH))
    for b in range(NBUF):
        write(0, b).wait()


@jax.jit
def _run(idx, table):
    n_total = idx.shape[0] * idx.shape[1] * idx.shape[2]
    mesh = plsc.VectorSubcoreMesh(core_axis_name="c", subcore_axis_name="s")
    k = functools.partial(
        pl.kernel,
        mesh=mesh,
        compiler_params=pltpu.CompilerParams(use_tc_tiling_on_sc=False),
        out_type=jax.ShapeDtypeStruct((n_total, DIM), jnp.float32),
        scratch_types=[pltpu.VMEM((idx.shape[1], CHUNK), jnp.int32)]
        + [pltpu.VMEM((CHUNK, DIM), jnp.float32) for _ in range(NBUF)]
        + [pltpu.SemaphoreType.DMA for _ in range(2 * NBUF)],
    )(_gather_body)
    return k(table, idx)


def kernel(input_data, embeddings_matrix):
    b, h = input_data.shape
    n_total = b * h
    assert n_total % (NUM_WORKERS * CHUNK) == 0
    n_chunk = n_total // (NUM_WORKERS * CHUNK)
    idx = input_data.astype(jnp.int32).reshape(NUM_WORKERS, n_chunk, CHUNK)
    out = _run(idx, embeddings_matrix)
    return out.reshape(b, h, DIM)
